# baseline (speedup 1.0000x reference)
import jax
import jax.numpy as jnp
from jax import lax
from jax.experimental import pallas as pl
from jax.experimental.pallas import tpu as pltpu
from jax.experimental.pallas import tpu_sc as plsc

_STEPS = 32


def _unit(x_hbm, o_hbm, ib, ob, isem, osem, wid, t, do_wait_out):
    g = wid * 64 + t
    b = lax.div(g, 1024)
    ho = lax.rem(lax.div(g, 32), 32)
    wo = lax.rem(g, 32)

    def in_copy(bb, hh, ww, sem):
        return pltpu.make_async_copy(
            x_hbm.at[bb, :, pl.ds(2 * hh, 2), pl.ds(2 * ww, 2)], ib, sem
        )

    in_copy(b, ho, wo, isem).wait()

    @pl.when(do_wait_out)
    def _():
        pltpu.make_async_copy(ob, o_hbm.at[b, ho, wo], osem).wait()

    lane = lax.iota(jnp.int32, 16)
    row_pat = 4 * (lane // 8) + 2 * ((lane // 4) % 2) + (lane // 2) % 2
    col_pat = lane % 2
    flat = ib.reshape(256, 64)
    rows = [row_pat + 8 * k for k in range(32)]

    @plsc.parallel_loop(0, 32, unroll=16)
    def _(zo):
        col = col_pat + 2 * zo
        for k in range(32):
            vec = plsc.load_gather(flat, [rows[k], col])
            ob[zo, pl.ds(16 * k, 16)] = vec

    pltpu.make_async_copy(ob, o_hbm.at[b, ho, wo], osem).start()

    t2 = t + 2

    @pl.when(t2 < 64)
    def _():
        g2 = wid * 64 + t2
        in_copy(
            lax.div(g2, 1024), lax.rem(lax.div(g2, 32), 32), lax.rem(g2, 32),
            isem,
        ).start()


def _sc_body(x_hbm, o_hbm, ib0, ib1, ob0, ob1, is0, is1, os0, os1):
    wid = lax.axis_index("s") * 2 + lax.axis_index("c")
    g0 = wid * 64

    pltpu.make_async_copy(
        x_hbm.at[lax.div(g0, 1024), :,
                 pl.ds(2 * lax.rem(lax.div(g0, 32), 32), 2),
                 pl.ds(2 * lax.rem(g0, 32), 2)],
        ib0, is0,
    ).start()
    g1 = g0 + 1
    pltpu.make_async_copy(
        x_hbm.at[lax.div(g1, 1024), :,
                 pl.ds(2 * lax.rem(lax.div(g1, 32), 32), 2),
                 pl.ds(2 * lax.rem(g1, 32), 2)],
        ib1, is1,
    ).start()

    @pl.loop(0, _STEPS)
    def _(j):
        t = 2 * j
        _unit(x_hbm, o_hbm, ib0, ob0, is0, os0, wid, t, j >= 1)
        _unit(x_hbm, o_hbm, ib1, ob1, is1, os1, wid, t + 1, j >= 1)

    gl = wid * 64 + 62
    bl = lax.div(gl, 1024)
    hl = lax.rem(lax.div(gl, 32), 32)
    pltpu.make_async_copy(ob0, o_hbm.at[bl, hl, lax.rem(gl, 32)], os0).wait()
    pltpu.make_async_copy(
        ob1, o_hbm.at[bl, hl, lax.rem(gl + 1, 32)], os1
    ).wait()


def kernel(x):
    B, C, H, W, Z = x.shape
    r = 2
    mesh = plsc.VectorSubcoreMesh(core_axis_name="c", subcore_axis_name="s")
    f = pl.kernel(
        _sc_body,
        out_type=jax.ShapeDtypeStruct(
            (B, H // r, W // r, Z // r, C * r**3), x.dtype
        ),
        mesh=mesh,
        compiler_params=pltpu.CompilerParams(needs_layout_passes=False),
        scratch_types=[
            pltpu.VMEM((C, r, r, Z), x.dtype),
            pltpu.VMEM((C, r, r, Z), x.dtype),
            pltpu.VMEM((Z // r, C * r**3), x.dtype),
            pltpu.VMEM((Z // r, C * r**3), x.dtype),
            pltpu.SemaphoreType.DMA,
            pltpu.SemaphoreType.DMA,
            pltpu.SemaphoreType.DMA,
            pltpu.SemaphoreType.DMA,
        ],
    )
    out = f(x)
    return jnp.transpose(out, (0, 4, 1, 2, 3))


# speedup vs baseline: 1.0204x; 1.0204x over previous
import jax
import jax.numpy as jnp
from jax import lax
from jax.experimental import pallas as pl
from jax.experimental.pallas import tpu as pltpu
from jax.experimental.pallas import tpu_sc as plsc

_STEPS = 32


def _unit(x_hbm, o_hbm, ib, ob, isem, osem, wid, t, do_wait_out):
    g = wid * 64 + t
    b = lax.div(g, 1024)
    ho = lax.rem(lax.div(g, 32), 32)
    wo = lax.rem(g, 32)

    def in_copy(bb, hh, ww, sem):
        return pltpu.make_async_copy(
            x_hbm.at[bb, :, pl.ds(2 * hh, 2), pl.ds(2 * ww, 2)], ib, sem
        )

    in_copy(b, ho, wo, isem).wait()

    @pl.when(do_wait_out)
    def _():
        pltpu.make_async_copy(ob, o_hbm.at[b, ho, wo], osem).wait()

    lane = lax.iota(jnp.int32, 16)
    row_pat = 4 * (lane // 8) + 2 * ((lane // 4) % 2) + (lane // 2) % 2
    col_pat = lane % 2
    flat = ib.reshape(256, 64)
    rows = [row_pat + 8 * k for k in range(32)]

    @plsc.parallel_loop(0, 32, unroll=8)
    def _(zo):
        col = col_pat + 2 * zo
        for k in range(32):
            vec = plsc.load_gather(flat, [rows[k], col])
            ob[zo, pl.ds(16 * k, 16)] = vec

    pltpu.make_async_copy(ob, o_hbm.at[b, ho, wo], osem).start()

    t2 = t + 2

    @pl.when(t2 < 64)
    def _():
        g2 = wid * 64 + t2
        in_copy(
            lax.div(g2, 1024), lax.rem(lax.div(g2, 32), 32), lax.rem(g2, 32),
            isem,
        ).start()


def _sc_body(x_hbm, o_hbm, ib0, ib1, ob0, ob1, is0, is1, os0, os1):
    wid = lax.axis_index("s") * 2 + lax.axis_index("c")
    g0 = wid * 64

    pltpu.make_async_copy(
        x_hbm.at[lax.div(g0, 1024), :,
                 pl.ds(2 * lax.rem(lax.div(g0, 32), 32), 2),
                 pl.ds(2 * lax.rem(g0, 32), 2)],
        ib0, is0,
    ).start()
    g1 = g0 + 1
    pltpu.make_async_copy(
        x_hbm.at[lax.div(g1, 1024), :,
                 pl.ds(2 * lax.rem(lax.div(g1, 32), 32), 2),
                 pl.ds(2 * lax.rem(g1, 32), 2)],
        ib1, is1,
    ).start()

    @pl.loop(0, _STEPS)
    def _(j):
        t = 2 * j
        _unit(x_hbm, o_hbm, ib0, ob0, is0, os0, wid, t, j >= 1)
        _unit(x_hbm, o_hbm, ib1, ob1, is1, os1, wid, t + 1, j >= 1)

    gl = wid * 64 + 62
    bl = lax.div(gl, 1024)
    hl = lax.rem(lax.div(gl, 32), 32)
    pltpu.make_async_copy(ob0, o_hbm.at[bl, hl, lax.rem(gl, 32)], os0).wait()
    pltpu.make_async_copy(
        ob1, o_hbm.at[bl, hl, lax.rem(gl + 1, 32)], os1
    ).wait()


def kernel(x):
    B, C, H, W, Z = x.shape
    r = 2
    mesh = plsc.VectorSubcoreMesh(core_axis_name="c", subcore_axis_name="s")
    f = pl.kernel(
        _sc_body,
        out_type=jax.ShapeDtypeStruct(
            (B, H // r, W // r, Z // r, C * r**3), x.dtype
        ),
        mesh=mesh,
        compiler_params=pltpu.CompilerParams(needs_layout_passes=False),
        scratch_types=[
            pltpu.VMEM((C, r, r, Z), x.dtype),
            pltpu.VMEM((C, r, r, Z), x.dtype),
            pltpu.VMEM((Z // r, C * r**3), x.dtype),
            pltpu.VMEM((Z // r, C * r**3), x.dtype),
            pltpu.SemaphoreType.DMA,
            pltpu.SemaphoreType.DMA,
            pltpu.SemaphoreType.DMA,
            pltpu.SemaphoreType.DMA,
        ],
    )
    out = f(x)
    return jnp.transpose(out, (0, 4, 1, 2, 3))
